# Initial kernel scaffold; baseline (speedup 1.0000x reference)
import jax
import jax.numpy as jnp
from jax.experimental import pallas as pl


def kernel(tokens, table):
    raise NotImplementedError("write your pallas kernel here")


# speedup vs baseline: 1.8308x; 1.8308x over previous
import functools

import jax
import jax.numpy as jnp
from jax import lax
from jax.experimental import pallas as pl
from jax.experimental.pallas import tpu as pltpu
from jax.experimental.pallas import tpu_sc as plsc

VOCAB = 32768
HIDDEN = 4096
NTOK = 2 * 8192

NC = 2
NS = 16
NW = NC * NS
PER_W = NTOK // NW
CH = 8
NCH = PER_W // CH
NBUF = 2


def _embed(idx_hbm, table_hbm, out_hbm, idx_v, buf_v, sem0, sem1):
    wid = lax.axis_index("s") * NC + lax.axis_index("c")
    base = wid * PER_W
    pltpu.sync_copy(idx_hbm.at[wid], idx_v)
    sems = (sem0, sem1)
    for b in range(NBUF):
        pltpu.async_copy(table_hbm.at[idx_v.at[b]], buf_v.at[b], sems[b])

    def body(i, carry):
        g = i * NBUF
        for b in range(NBUF):
            c = g + b
            pltpu.make_async_copy(
                table_hbm.at[idx_v.at[c]], buf_v.at[b], sems[b]).wait()
            pltpu.sync_copy(buf_v.at[b], out_hbm.at[pl.ds(base + c * CH, CH)])

            @pl.when(c + NBUF < NCH)
            def _():
                pltpu.async_copy(
                    table_hbm.at[idx_v.at[c + NBUF]], buf_v.at[b], sems[b])
        return carry

    lax.fori_loop(0, NCH // NBUF, body, 0)


@jax.jit
def kernel(tokens, table):
    idx = tokens.astype(jnp.int32).reshape(NW, NCH, CH)
    mesh = plsc.VectorSubcoreMesh(core_axis_name="c", subcore_axis_name="s")
    emb = functools.partial(
        pl.kernel,
        mesh=mesh,
        out_type=jax.ShapeDtypeStruct((NTOK, HIDDEN), jnp.float32),
        scratch_types=[
            pltpu.VMEM((NCH, CH), jnp.int32),
            pltpu.VMEM((NBUF, CH, HIDDEN), jnp.float32),
            pltpu.SemaphoreType.DMA,
            pltpu.SemaphoreType.DMA,
        ],
    )(_embed)
    out = emb(idx, table)
    return out.reshape(2, 8192, HIDDEN)
